# baseline (speedup 1.0000x reference)
import jax
import jax.numpy as jnp
from jax.experimental import pallas as pl

_NB = 512


def _chamfer_kernel(x_ref, yt_ref, d1_ref, d2_ref):
    x = x_ref[0]
    yt = yt_ref[0]
    inner = jax.lax.dot(x, yt, preferred_element_type=jnp.float32)
    x2 = jnp.sum(x * x, axis=1, keepdims=True)
    y2 = jnp.sum(yt * yt, axis=0, keepdims=True)
    d = (x2 + y2) - 2.0 * inner

    d1_ref[0, 0, :] = jnp.maximum(jnp.min(d, axis=1), 0.0)

    colmin = jnp.maximum(jnp.min(d, axis=0), 0.0)
    i = pl.program_id(1)

    @pl.when(i == 0)
    def _init():
        d2_ref[0, 0, :] = colmin

    @pl.when(i > 0)
    def _acc():
        d2_ref[0, 0, :] = jnp.minimum(d2_ref[0, 0, :], colmin)


def kernel(input1, input2):
    B, N, D = input1.shape
    M = input2.shape[1]
    yt = jnp.transpose(input2, (0, 2, 1))
    grid = (B, N // _NB)
    d1, d2 = pl.pallas_call(
        _chamfer_kernel,
        grid=grid,
        in_specs=[
            pl.BlockSpec((1, _NB, D), lambda b, i: (b, i, 0)),
            pl.BlockSpec((1, D, M), lambda b, i: (b, 0, 0)),
        ],
        out_specs=[
            pl.BlockSpec((1, 1, _NB), lambda b, i: (b * (N // _NB) + i, 0, 0)),
            pl.BlockSpec((1, 1, M), lambda b, i: (b, 0, 0)),
        ],
        out_shape=[
            jax.ShapeDtypeStruct((B * (N // _NB), 1, _NB), jnp.float32),
            jax.ShapeDtypeStruct((B, 1, M), jnp.float32),
        ],
    )(input1, yt)
    return (d1.reshape(B, N), d2.reshape(B, M))


# speedup vs baseline: 1.1436x; 1.1436x over previous
import jax
import jax.numpy as jnp
from jax.experimental import pallas as pl

_NB = 512


def _chamfer_kernel(x_ref, yt_ref, d1_ref, d2_ref):
    x = x_ref[0]
    yt = yt_ref[0]
    x2 = jnp.sum(x * x, axis=1, keepdims=True)
    y2 = jnp.sum(yt * yt, axis=0, keepdims=True)
    ones_x = jnp.ones_like(x2)
    ones_y = jnp.ones_like(y2)
    xa = jnp.concatenate([x, x2, ones_x], axis=1)
    ya = jnp.concatenate([-2.0 * yt, ones_y, y2], axis=0)
    d = jax.lax.dot(xa, ya, preferred_element_type=jnp.float32)

    d1_ref[0, 0, :] = jnp.maximum(jnp.min(d, axis=1), 0.0)

    colmin = jnp.maximum(jnp.min(d, axis=0), 0.0)
    i = pl.program_id(1)

    @pl.when(i == 0)
    def _init():
        d2_ref[0, 0, :] = colmin

    @pl.when(i > 0)
    def _acc():
        d2_ref[0, 0, :] = jnp.minimum(d2_ref[0, 0, :], colmin)


def kernel(input1, input2):
    B, N, D = input1.shape
    M = input2.shape[1]
    yt = jnp.transpose(input2, (0, 2, 1))
    grid = (B, N // _NB)
    d1, d2 = pl.pallas_call(
        _chamfer_kernel,
        grid=grid,
        in_specs=[
            pl.BlockSpec((1, _NB, D), lambda b, i: (b, i, 0)),
            pl.BlockSpec((1, D, M), lambda b, i: (b, 0, 0)),
        ],
        out_specs=[
            pl.BlockSpec((1, 1, _NB), lambda b, i: (b * (N // _NB) + i, 0, 0)),
            pl.BlockSpec((1, 1, M), lambda b, i: (b, 0, 0)),
        ],
        out_shape=[
            jax.ShapeDtypeStruct((B * (N // _NB), 1, _NB), jnp.float32),
            jax.ShapeDtypeStruct((B, 1, M), jnp.float32),
        ],
    )(input1, yt)
    return (d1.reshape(B, N), d2.reshape(B, M))
